# baseline (speedup 1.0000x reference)
import functools

import jax
import jax.numpy as jnp
from jax import lax
from jax.experimental import pallas as pl
from jax.experimental.pallas import tpu as pltpu
from jax.experimental.pallas import tpu_sc as plsc

E = 64
K = 2
H = 1024
F = 512
T = 2048

M = 128
G = 96
P = G * M

_TB = 128
_NTB = T // _TB

_NC = 2
_NS = 16
_NW = _NC * _NS
_L = 16


def _router_body(x_ref, gw_ref, w1_ref, w2_ref, i1_ref, i2_ref,
                 r1_ref, r2_ref, cnt_ref, meta_ref, run_scr):
    b = pl.program_id(0)

    @pl.when(b == 0)
    def _():
        run_scr[...] = jnp.zeros_like(run_scr)

    logits = jnp.dot(x_ref[...], gw_ref[...], preferred_element_type=jnp.float32)
    p = jax.nn.softmax(logits, axis=-1)
    lane = lax.broadcasted_iota(jnp.int32, (_TB, E), 1)

    m1 = jnp.max(p, axis=-1, keepdims=True)
    i1 = jnp.min(jnp.where(p >= m1, lane, E), axis=-1, keepdims=True)
    p2 = jnp.where(lane == i1, -1.0, p)
    m2 = jnp.max(p2, axis=-1, keepdims=True)
    i2 = jnp.min(jnp.where(p2 >= m2, lane, E), axis=-1, keepdims=True)
    s = m1 + m2

    oh1 = (lane == i1).astype(jnp.float32)
    oh2 = (lane == i2).astype(jnp.float32)
    oh = oh1 + oh2

    row_i = lax.broadcasted_iota(jnp.int32, (_TB, _TB), 0)
    col_j = lax.broadcasted_iota(jnp.int32, (_TB, _TB), 1)
    tri = (col_j < row_i).astype(jnp.float32)
    cum = run_scr[0:1, :] + jnp.dot(tri, oh, preferred_element_type=jnp.float32)

    w1_ref[...] = m1 / s
    w2_ref[...] = m2 / s
    i1_ref[...] = i1.astype(jnp.float32)
    i2_ref[...] = i2.astype(jnp.float32)
    r1_ref[...] = jnp.sum(cum * oh1, axis=-1, keepdims=True)
    r2_ref[...] = jnp.sum(cum * oh2, axis=-1, keepdims=True)

    run_scr[0:1, :] += jnp.sum(oh, axis=0, keepdims=True)
    cnt_ref[...] = jnp.broadcast_to(run_scr[0:1, :], (8, E))

    @pl.when(b == _NTB - 1)
    def _meta():
        _compute_meta(run_scr, meta_ref)


def _compute_meta(run_scr, meta_ref):
    cnt_row = run_scr[0:1, :]
    nt = jnp.floor((cnt_row + (M - 1)) * (1.0 / M))
    er = lax.broadcasted_iota(jnp.int32, (E, E), 0)
    ec = lax.broadcasted_iota(jnp.int32, (E, E), 1)
    tri_le = (er <= ec).astype(jnp.float32)
    cum_row = jnp.dot(nt, tri_le, preferred_element_type=jnp.float32)
    base_row = (cum_row - nt) * float(M)
    used = cum_row[:, E - 1:E]
    cum_col = lax.dot_general(
        (er >= ec).astype(jnp.float32), nt,
        (((1,), (1,)), ((), ())),
        preferred_element_type=jnp.float32)
    colg = lax.broadcasted_iota(jnp.int32, (E, 2 * E), 1).astype(jnp.float32)
    ge = (cum_col <= colg).astype(jnp.float32)
    eid_row = jnp.dot(jnp.ones((1, E), jnp.float32), ge,
                      preferred_element_type=jnp.float32)
    eid_row = jnp.minimum(eid_row, float(E - 1))
    meta_ref[...] = jnp.concatenate(
        [eid_row,
         jnp.concatenate([base_row, jnp.zeros((1, E), jnp.float32)], axis=1),
         jnp.broadcast_to(used, (1, 2 * E)),
         jnp.zeros((5, 2 * E), jnp.float32)], axis=0)


def _router(x, gw):
    col = jax.ShapeDtypeStruct((T, 1), jnp.float32)
    return pl.pallas_call(
        _router_body,
        grid=(_NTB,),
        in_specs=[
            pl.BlockSpec((_TB, H), lambda b: (b, 0)),
            pl.BlockSpec((H, E), lambda b: (0, 0)),
        ],
        out_specs=[pl.BlockSpec((_TB, 1), lambda b: (b, 0))] * 6
        + [pl.BlockSpec((8, E), lambda b: (0, 0)),
           pl.BlockSpec((8, 2 * E), lambda b: (0, 0))],
        out_shape=[col] * 6 + [jax.ShapeDtypeStruct((8, E), jnp.float32),
                               jax.ShapeDtypeStruct((8, 2 * E), jnp.float32)],
        scratch_shapes=[pltpu.VMEM((8, E), jnp.float32)],
    )(x, gw)


def _sc_dispatch(x, i1f, i2f, r1f, r2f, meta):
    tok_per_w = T // _NW
    rows_per_w = P // _NW
    chunk = 48
    mesh = plsc.VectorSubcoreMesh(core_axis_name="c", subcore_axis_name="s")

    @functools.partial(
        pl.kernel, mesh=mesh,
        compiler_params=pltpu.CompilerParams(needs_layout_passes=False),
        out_type=(
            jax.ShapeDtypeStruct((P, H), jnp.float32),
            jax.ShapeDtypeStruct((T,), jnp.int32),
            jax.ShapeDtypeStruct((T,), jnp.int32),
            jax.ShapeDtypeStruct((G,), jnp.int32),
            jax.ShapeDtypeStruct((16,), jnp.int32),
        ),
        scratch_types=[
            pltpu.VMEM((2 * E,), jnp.float32),
            pltpu.VMEM((E,), jnp.int32),
            pltpu.VMEM((T,), jnp.float32),
            pltpu.VMEM((T,), jnp.float32),
            pltpu.VMEM((T,), jnp.float32),
            pltpu.VMEM((T,), jnp.float32),
            pltpu.VMEM((tok_per_w,), jnp.int32),
            pltpu.VMEM((tok_per_w,), jnp.int32),
            pltpu.VMEM((rows_per_w,), jnp.int32),
            pltpu.VMEM((G,), jnp.int32),
            pltpu.VMEM((16,), jnp.int32),
            pltpu.VMEM((chunk, H), jnp.float32),
            pltpu.VMEM((chunk, H), jnp.float32),
            pltpu.SemaphoreType.DMA,
            pltpu.SemaphoreType.DMA,
            pltpu.SemaphoreType.DMA,
            pltpu.SemaphoreType.DMA,
        ],
    )
    def k(x_hbm, i1_hbm, i2_hbm, r1_hbm, r2_hbm, meta_hbm,
          xs_hbm, p1_hbm, p2_hbm, eid_hbm, used_hbm,
          mrow_v, base_v, i1_v, i2_v, r1_v, r2_v,
          pos1_v, pos2_v, src_v, eid_v, used_v, b0, b1, sem0, sem1, wsem0, wsem1):
        wid = lax.axis_index("s") * _NC + lax.axis_index("c")
        tok0 = wid * tok_per_w
        row0 = wid * rows_per_w

        pltpu.sync_copy(i1_hbm, i1_v)
        pltpu.sync_copy(i2_hbm, i2_v)
        pltpu.sync_copy(r1_hbm, r1_v)
        pltpu.sync_copy(r2_hbm, r2_v)

        pltpu.sync_copy(meta_hbm.at[1], mrow_v)
        for v in range(E // _L):
            base_v[pl.ds(v * _L, _L)] = mrow_v[pl.ds(v * _L, _L)].astype(jnp.int32)

        def initv(v, carry):
            src_v[pl.ds(v * _L, _L)] = (
                lax.iota(jnp.int32, _L) + (row0 + v * _L)) & (T - 1)
            return carry

        lax.fori_loop(0, rows_per_w // _L, initv, 0, unroll=4)

        def scan(v, carry):
            sl = pl.ds(v * _L, _L)
            tok16 = lax.iota(jnp.int32, _L) + v * _L
            e16 = i1_v[sl].astype(jnp.int32)
            p16 = plsc.load_gather(base_v, [e16]) + r1_v[sl].astype(jnp.int32)
            rel1 = p16 - row0
            plsc.store_scatter(src_v, [rel1], tok16,
                               mask=(rel1 >= 0) & (rel1 < rows_per_w))
            f16 = i2_v[sl].astype(jnp.int32)
            q16 = plsc.load_gather(base_v, [f16]) + r2_v[sl].astype(jnp.int32)
            rel2 = q16 - row0
            plsc.store_scatter(src_v, [rel2], tok16,
                               mask=(rel2 >= 0) & (rel2 < rows_per_w))

            @pl.when(v // (tok_per_w // _L) == wid)
            def _():
                u = v % (tok_per_w // _L)
                usl = pl.ds(u * _L, _L)
                pos1_v[usl] = p16
                pos2_v[usl] = q16

            return carry

        lax.fori_loop(0, T // _L, scan, 0, unroll=False)
        pltpu.sync_copy(pos1_v, p1_hbm.at[pl.ds(tok0, tok_per_w)])
        pltpu.sync_copy(pos2_v, p2_hbm.at[pl.ds(tok0, tok_per_w)])

        @pl.when(wid == 0)
        def _():
            pltpu.sync_copy(meta_hbm.at[0], mrow_v)
            for v in range(G // _L):
                sl = pl.ds(v * _L, _L)
                eid_v[sl] = mrow_v[sl].astype(jnp.int32)
            pltpu.sync_copy(eid_v, eid_hbm)

        pltpu.sync_copy(meta_hbm.at[2], mrow_v)
        uf16 = mrow_v[pl.ds(0, _L)]
        used_s = uf16[0].astype(jnp.int32)

        @pl.when(wid == 0)
        def _():
            used_v[...] = uf16.astype(jnp.int32)
            pltpu.sync_copy(used_v, used_hbm)

        cnt_rows = jnp.clip(used_s * M - row0, 0, rows_per_w)
        nch = (cnt_rows + chunk - 1) // chunk

        def pair(cp, carry):
            c0 = 2 * cp
            c1 = c0 + 1
            d0 = pltpu.make_async_copy(
                x_hbm.at[src_v.at[pl.ds(c0 * chunk, chunk)]], b0, sem0)
            d1 = pltpu.make_async_copy(
                x_hbm.at[src_v.at[pl.ds(c1 * chunk, chunk)]], b1, sem1)
            w0 = pltpu.make_async_copy(
                b0, xs_hbm.at[pl.ds(row0 + c0 * chunk, chunk)], wsem0)
            w1 = pltpu.make_async_copy(
                b1, xs_hbm.at[pl.ds(row0 + c1 * chunk, chunk)], wsem1)
            pw0 = pltpu.make_async_copy(
                b0, xs_hbm.at[pl.ds(row0 + (c0 - 2) * chunk, chunk)], wsem0)
            pw1 = pltpu.make_async_copy(
                b1, xs_hbm.at[pl.ds(row0 + (c1 - 2) * chunk, chunk)], wsem1)

            @pl.when((cp > 0) & (c0 < nch))
            def _():
                pw0.wait()

            @pl.when(c0 < nch)
            def _():
                d0.start()

            @pl.when((cp > 0) & (c1 < nch))
            def _():
                pw1.wait()

            @pl.when(c1 < nch)
            def _():
                d1.start()

            @pl.when(c0 < nch)
            def _():
                d0.wait()
                w0.start()

            @pl.when(c1 < nch)
            def _():
                d1.wait()
                w1.start()

            return carry

        lax.fori_loop(0, (nch + 1) // 2, pair, 0, unroll=False)

        @pl.when(nch >= 1)
        def _():
            lastc = nch - 1
            pltpu.make_async_copy(
                b0, xs_hbm.at[pl.ds(row0 + lastc * chunk, chunk)], wsem0).wait()

        @pl.when(nch >= 2)
        def _():
            lastc = nch - 2
            pltpu.make_async_copy(
                b1, xs_hbm.at[pl.ds(row0 + lastc * chunk, chunk)], wsem1).wait()

    return k(x, i1f, i2f, r1f, r2f, meta)


def _gmm_body(eid_ref, used_ref, xs_ref, guw_ref, dw_ref, out_ref):
    g = pl.program_id(0)

    @pl.when(g < used_ref[0])
    def _():
        xs = xs_ref[...]
        gu = jnp.dot(xs, guw_ref[0], preferred_element_type=jnp.float32)
        a = gu[:, :F]
        u = gu[:, F:]
        h = a * jax.nn.sigmoid(a) * u
        out_ref[...] = jnp.dot(h, dw_ref[0], preferred_element_type=jnp.float32)


def _gmm(eid, used, xs, guw, dw):
    grid_spec = pltpu.PrefetchScalarGridSpec(
        num_scalar_prefetch=2,
        grid=(G,),
        in_specs=[
            pl.BlockSpec((M, H), lambda g, eid, u: (jnp.minimum(g, u[0] - 1), 0)),
            pl.BlockSpec((1, H, 2 * F), lambda g, eid, u: (eid[g], 0, 0)),
            pl.BlockSpec((1, F, H), lambda g, eid, u: (eid[g], 0, 0)),
        ],
        out_specs=pl.BlockSpec(
            (M, H), lambda g, eid, u: (jnp.minimum(g, u[0] - 1), 0)),
    )
    return pl.pallas_call(
        _gmm_body,
        grid_spec=grid_spec,
        out_shape=jax.ShapeDtypeStruct((P, H), jnp.float32),
    )(eid, used, xs, guw, dw)


def _sc_combine(out_sorted, pos1, pos2, w1, w2):
    tok_per_w = T // _NW
    chunk = 16
    nchunks = tok_per_w // chunk
    hvecs = H // _L
    mesh = plsc.VectorSubcoreMesh(core_axis_name="c", subcore_axis_name="s")

    @functools.partial(
        pl.kernel, mesh=mesh,
        compiler_params=pltpu.CompilerParams(needs_layout_passes=False),
        out_type=jax.ShapeDtypeStruct((T, H), jnp.float32),
        scratch_types=[
            pltpu.VMEM((tok_per_w,), jnp.int32),
            pltpu.VMEM((tok_per_w,), jnp.int32),
            pltpu.VMEM((2 * tok_per_w,), jnp.int32),
            pltpu.VMEM((tok_per_w,), jnp.float32),
            pltpu.VMEM((tok_per_w,), jnp.float32),
            pltpu.VMEM((2 * chunk, H), jnp.float32),
            pltpu.VMEM((2 * chunk, H), jnp.float32),
            pltpu.SemaphoreType.DMA,
            pltpu.SemaphoreType.DMA,
        ],
    )
    def k(os_hbm, p1_hbm, p2_hbm, w1_hbm, w2_hbm, out_hbm,
          i1_v, i2_v, ix_v, w1_v, w2_v, bA, bB, semA, semB):
        wid = lax.axis_index("s") * _NC + lax.axis_index("c")
        base = wid * tok_per_w
        pltpu.sync_copy(p1_hbm.at[pl.ds(base, tok_per_w)], i1_v)
        pltpu.sync_copy(p2_hbm.at[pl.ds(base, tok_per_w)], i2_v)
        pltpu.sync_copy(w1_hbm.at[pl.ds(base, tok_per_w)], w1_v)
        pltpu.sync_copy(w2_hbm.at[pl.ds(base, tok_per_w)], w2_v)

        for c in range(nchunks):
            ix_v[pl.ds(2 * chunk * c, _L)] = i1_v[pl.ds(chunk * c, _L)]
            ix_v[pl.ds(2 * chunk * c + chunk, _L)] = i2_v[pl.ds(chunk * c, _L)]

        def body2(cp, carry):
            c0 = 2 * cp
            c1 = c0 + 1
            dA = pltpu.make_async_copy(
                os_hbm.at[ix_v.at[pl.ds(2 * chunk * c0, 2 * chunk)]], bA, semA)
            dB = pltpu.make_async_copy(
                os_hbm.at[ix_v.at[pl.ds(2 * chunk * c1, 2 * chunk)]], bB, semB)
            dA.start()
            dB.start()

            def work(buf, c, sem_wait):
                sem_wait.wait()

                def row(r, rc):
                    ridx = jnp.full((_L,), c * chunk + r, jnp.int32)
                    ws1 = plsc.load_gather(w1_v, [ridx])
                    ws2 = plsc.load_gather(w2_v, [ridx])

                    def add(j, jc):
                        slj = pl.ds(j * _L, _L)
                        buf[r, slj] = ws1 * buf[r, slj] + ws2 * buf[r + chunk, slj]
                        return jc

                    lax.fori_loop(0, hvecs, add, 0, unroll=4)
                    return rc

                lax.fori_loop(0, chunk, row, 0, unroll=False)
                pltpu.sync_copy(
                    buf.at[pl.ds(0, chunk)],
                    out_hbm.at[pl.ds(base + c * chunk, chunk)])

            work(bA, c0, dA)
            work(bB, c1, dB)
            return carry

        lax.fori_loop(0, nchunks // 2, body2, 0, unroll=False)

    return k(out_sorted, pos1, pos2, w1, w2)


def kernel(hidden_states, gate_weight, gate_up_weight, down_weight):
    x = hidden_states
    w1, w2, i1f, i2f, r1f, r2f, cnt8, meta = _router(x, gate_weight)

    xs, pos1, pos2, eid, used = _sc_dispatch(
        x, i1f.reshape(T), i2f.reshape(T), r1f.reshape(T), r2f.reshape(T), meta)
    out_sorted = _gmm(eid, used, xs, gate_up_weight, down_weight)
    return _sc_combine(out_sorted, pos1, pos2, w1.reshape(T), w2.reshape(T))


# speedup vs baseline: 1.0763x; 1.0763x over previous
import functools

import jax
import jax.numpy as jnp
from jax import lax
from jax.experimental import pallas as pl
from jax.experimental.pallas import tpu as pltpu
from jax.experimental.pallas import tpu_sc as plsc

E = 64
K = 2
H = 1024
F = 512
T = 2048

M = 128
G = 96
P = G * M

_TB = 128
_NTB = T // _TB

_NC = 2
_NS = 16
_NW = _NC * _NS
_L = 16


def _router_body(x_ref, gw_ref, w1_ref, w2_ref, i1_ref, i2_ref,
                 r1_ref, r2_ref, cnt_ref, meta_ref, run_scr):
    b = pl.program_id(0)

    @pl.when(b == 0)
    def _():
        run_scr[...] = jnp.zeros_like(run_scr)

    logits = jnp.dot(x_ref[...], gw_ref[...], preferred_element_type=jnp.float32)
    p = jax.nn.softmax(logits, axis=-1)
    lane = lax.broadcasted_iota(jnp.int32, (_TB, E), 1)

    m1 = jnp.max(p, axis=-1, keepdims=True)
    i1 = jnp.min(jnp.where(p >= m1, lane, E), axis=-1, keepdims=True)
    p2 = jnp.where(lane == i1, -1.0, p)
    m2 = jnp.max(p2, axis=-1, keepdims=True)
    i2 = jnp.min(jnp.where(p2 >= m2, lane, E), axis=-1, keepdims=True)
    s = m1 + m2

    oh1 = (lane == i1).astype(jnp.float32)
    oh2 = (lane == i2).astype(jnp.float32)
    oh = oh1 + oh2

    row_i = lax.broadcasted_iota(jnp.int32, (_TB, _TB), 0)
    col_j = lax.broadcasted_iota(jnp.int32, (_TB, _TB), 1)
    tri = (col_j < row_i).astype(jnp.float32)
    cum = run_scr[0:1, :] + jnp.dot(tri, oh, preferred_element_type=jnp.float32)

    w1_ref[...] = m1 / s
    w2_ref[...] = m2 / s
    i1_ref[...] = i1.astype(jnp.float32)
    i2_ref[...] = i2.astype(jnp.float32)
    r1_ref[...] = jnp.sum(cum * oh1, axis=-1, keepdims=True)
    r2_ref[...] = jnp.sum(cum * oh2, axis=-1, keepdims=True)

    run_scr[0:1, :] += jnp.sum(oh, axis=0, keepdims=True)
    cnt_ref[...] = jnp.broadcast_to(run_scr[0:1, :], (8, E))

    @pl.when(b == _NTB - 1)
    def _meta():
        _compute_meta(run_scr, meta_ref)


def _compute_meta(run_scr, meta_ref):
    cnt_row = run_scr[0:1, :]
    nt = jnp.floor((cnt_row + (M - 1)) * (1.0 / M))
    er = lax.broadcasted_iota(jnp.int32, (E, E), 0)
    ec = lax.broadcasted_iota(jnp.int32, (E, E), 1)
    tri_le = (er <= ec).astype(jnp.float32)
    cum_row = jnp.dot(nt, tri_le, preferred_element_type=jnp.float32)
    base_row = (cum_row - nt) * float(M)
    used = cum_row[:, E - 1:E]
    cum_col = lax.dot_general(
        (er >= ec).astype(jnp.float32), nt,
        (((1,), (1,)), ((), ())),
        preferred_element_type=jnp.float32)
    colg = lax.broadcasted_iota(jnp.int32, (E, 2 * E), 1).astype(jnp.float32)
    ge = (cum_col <= colg).astype(jnp.float32)
    eid_row = jnp.dot(jnp.ones((1, E), jnp.float32), ge,
                      preferred_element_type=jnp.float32)
    eid_row = jnp.minimum(eid_row, float(E - 1))
    meta_ref[...] = jnp.concatenate(
        [eid_row,
         jnp.concatenate([base_row, jnp.zeros((1, E), jnp.float32)], axis=1),
         jnp.broadcast_to(used, (1, 2 * E)),
         jnp.zeros((5, 2 * E), jnp.float32)], axis=0)


def _router(x, gw):
    col = jax.ShapeDtypeStruct((T, 1), jnp.float32)
    return pl.pallas_call(
        _router_body,
        grid=(_NTB,),
        in_specs=[
            pl.BlockSpec((_TB, H), lambda b: (b, 0)),
            pl.BlockSpec((H, E), lambda b: (0, 0)),
        ],
        out_specs=[pl.BlockSpec((_TB, 1), lambda b: (b, 0))] * 6
        + [pl.BlockSpec((8, E), lambda b: (0, 0)),
           pl.BlockSpec((8, 2 * E), lambda b: (0, 0))],
        out_shape=[col] * 6 + [jax.ShapeDtypeStruct((8, E), jnp.float32),
                               jax.ShapeDtypeStruct((8, 2 * E), jnp.float32)],
        scratch_shapes=[pltpu.VMEM((8, E), jnp.float32)],
    )(x, gw)


def _sc_dispatch(x, i1f, i2f, r1f, r2f, meta):
    tok_per_w = T // _NW
    rows_per_w = P // _NW
    chunk = 48
    mesh = plsc.VectorSubcoreMesh(core_axis_name="c", subcore_axis_name="s")

    @functools.partial(
        pl.kernel, mesh=mesh,
        compiler_params=pltpu.CompilerParams(needs_layout_passes=False),
        out_type=(
            jax.ShapeDtypeStruct((P, H), jnp.float32),
            jax.ShapeDtypeStruct((T,), jnp.int32),
            jax.ShapeDtypeStruct((T,), jnp.int32),
            jax.ShapeDtypeStruct((G,), jnp.int32),
            jax.ShapeDtypeStruct((16,), jnp.int32),
        ),
        scratch_types=[
            pltpu.VMEM((2 * E,), jnp.float32),
            pltpu.VMEM((E,), jnp.int32),
            pltpu.VMEM((T,), jnp.float32),
            pltpu.VMEM((T,), jnp.float32),
            pltpu.VMEM((T,), jnp.float32),
            pltpu.VMEM((T,), jnp.float32),
            pltpu.VMEM((tok_per_w,), jnp.int32),
            pltpu.VMEM((tok_per_w,), jnp.int32),
            pltpu.VMEM((rows_per_w,), jnp.int32),
            pltpu.VMEM((G,), jnp.int32),
            pltpu.VMEM((16,), jnp.int32),
            pltpu.VMEM((chunk, H), jnp.float32),
            pltpu.VMEM((chunk, H), jnp.float32),
            pltpu.SemaphoreType.DMA,
            pltpu.SemaphoreType.DMA,
            pltpu.SemaphoreType.DMA,
            pltpu.SemaphoreType.DMA,
        ],
    )
    def k(x_hbm, i1_hbm, i2_hbm, r1_hbm, r2_hbm, meta_hbm,
          xs_hbm, p1_hbm, p2_hbm, eid_hbm, used_hbm,
          mrow_v, base_v, i1_v, i2_v, r1_v, r2_v,
          pos1_v, pos2_v, src_v, eid_v, used_v, b0, b1, sem0, sem1, wsem0, wsem1):
        wid = lax.axis_index("s") * _NC + lax.axis_index("c")
        tok0 = wid * tok_per_w
        row0 = wid * rows_per_w

        pltpu.sync_copy(i1_hbm, i1_v)
        pltpu.sync_copy(i2_hbm, i2_v)
        pltpu.sync_copy(r1_hbm, r1_v)
        pltpu.sync_copy(r2_hbm, r2_v)

        pltpu.sync_copy(meta_hbm.at[1], mrow_v)
        for v in range(E // _L):
            base_v[pl.ds(v * _L, _L)] = mrow_v[pl.ds(v * _L, _L)].astype(jnp.int32)

        def initv(v, carry):
            src_v[pl.ds(v * _L, _L)] = (
                lax.iota(jnp.int32, _L) + (row0 + v * _L)) & (T - 1)
            return carry

        lax.fori_loop(0, rows_per_w // _L, initv, 0, unroll=4)

        def scan(v, carry):
            sl = pl.ds(v * _L, _L)
            tok16 = lax.iota(jnp.int32, _L) + v * _L
            e16 = i1_v[sl].astype(jnp.int32)
            p16 = plsc.load_gather(base_v, [e16]) + r1_v[sl].astype(jnp.int32)
            rel1 = p16 - row0
            plsc.store_scatter(src_v, [rel1], tok16,
                               mask=(rel1 >= 0) & (rel1 < rows_per_w))
            f16 = i2_v[sl].astype(jnp.int32)
            q16 = plsc.load_gather(base_v, [f16]) + r2_v[sl].astype(jnp.int32)
            rel2 = q16 - row0
            plsc.store_scatter(src_v, [rel2], tok16,
                               mask=(rel2 >= 0) & (rel2 < rows_per_w))

            @pl.when(v // (tok_per_w // _L) == wid)
            def _():
                u = v % (tok_per_w // _L)
                usl = pl.ds(u * _L, _L)
                pos1_v[usl] = p16
                pos2_v[usl] = q16

            return carry

        lax.fori_loop(0, T // _L, scan, 0, unroll=False)
        pltpu.sync_copy(pos1_v, p1_hbm.at[pl.ds(tok0, tok_per_w)])
        pltpu.sync_copy(pos2_v, p2_hbm.at[pl.ds(tok0, tok_per_w)])

        @pl.when(wid == 0)
        def _():
            pltpu.sync_copy(meta_hbm.at[0], mrow_v)
            for v in range(G // _L):
                sl = pl.ds(v * _L, _L)
                eid_v[sl] = mrow_v[sl].astype(jnp.int32)
            pltpu.sync_copy(eid_v, eid_hbm)

        pltpu.sync_copy(meta_hbm.at[2], mrow_v)
        uf16 = mrow_v[pl.ds(0, _L)]
        used_s = uf16[0].astype(jnp.int32)

        @pl.when(wid == 0)
        def _():
            used_v[...] = uf16.astype(jnp.int32)
            pltpu.sync_copy(used_v, used_hbm)

        cnt_rows = jnp.clip(used_s * M - row0, 0, rows_per_w)
        nch = (cnt_rows + chunk - 1) // chunk

        def pair(cp, carry):
            c0 = 2 * cp
            c1 = c0 + 1
            d0 = pltpu.make_async_copy(
                x_hbm.at[src_v.at[pl.ds(c0 * chunk, chunk)]], b0, sem0)
            d1 = pltpu.make_async_copy(
                x_hbm.at[src_v.at[pl.ds(c1 * chunk, chunk)]], b1, sem1)
            w0 = pltpu.make_async_copy(
                b0, xs_hbm.at[pl.ds(row0 + c0 * chunk, chunk)], wsem0)
            w1 = pltpu.make_async_copy(
                b1, xs_hbm.at[pl.ds(row0 + c1 * chunk, chunk)], wsem1)
            pw0 = pltpu.make_async_copy(
                b0, xs_hbm.at[pl.ds(row0 + (c0 - 2) * chunk, chunk)], wsem0)
            pw1 = pltpu.make_async_copy(
                b1, xs_hbm.at[pl.ds(row0 + (c1 - 2) * chunk, chunk)], wsem1)

            @pl.when((cp > 0) & (c0 < nch))
            def _():
                pw0.wait()

            @pl.when(c0 < nch)
            def _():
                d0.start()

            @pl.when((cp > 0) & (c1 < nch))
            def _():
                pw1.wait()

            @pl.when(c1 < nch)
            def _():
                d1.start()

            @pl.when(c0 < nch)
            def _():
                d0.wait()
                w0.start()

            @pl.when(c1 < nch)
            def _():
                d1.wait()
                w1.start()

            return carry

        lax.fori_loop(0, (nch + 1) // 2, pair, 0, unroll=False)

        @pl.when(nch >= 1)
        def _():
            lastc = nch - 1
            pltpu.make_async_copy(
                b0, xs_hbm.at[pl.ds(row0 + lastc * chunk, chunk)], wsem0).wait()

        @pl.when(nch >= 2)
        def _():
            lastc = nch - 2
            pltpu.make_async_copy(
                b1, xs_hbm.at[pl.ds(row0 + lastc * chunk, chunk)], wsem1).wait()

    return k(x, i1f, i2f, r1f, r2f, meta)


def _gmm_body(eid_ref, used_ref, xs_ref, guw_ref, dw_ref, out_ref):
    g = pl.program_id(0)

    @pl.when(g < used_ref[0])
    def _():
        xs = xs_ref[...]
        gu = jnp.dot(xs, guw_ref[0], preferred_element_type=jnp.float32)
        a = gu[:, :F]
        u = gu[:, F:]
        h = a * jax.nn.sigmoid(a) * u
        out_ref[...] = jnp.dot(h, dw_ref[0], preferred_element_type=jnp.float32)


def _gmm(eid, used, xs, guw, dw):
    grid_spec = pltpu.PrefetchScalarGridSpec(
        num_scalar_prefetch=2,
        grid=(G,),
        in_specs=[
            pl.BlockSpec((M, H), lambda g, eid, u: (jnp.minimum(g, u[0] - 1), 0)),
            pl.BlockSpec((1, H, 2 * F), lambda g, eid, u: (eid[g], 0, 0)),
            pl.BlockSpec((1, F, H), lambda g, eid, u: (eid[g], 0, 0)),
        ],
        out_specs=pl.BlockSpec(
            (M, H), lambda g, eid, u: (jnp.minimum(g, u[0] - 1), 0)),
    )
    return pl.pallas_call(
        _gmm_body,
        grid_spec=grid_spec,
        out_shape=jax.ShapeDtypeStruct((P, H), jnp.float32),
    )(eid, used, xs, guw, dw)


def _sc_combine(out_sorted, pos1, pos2, w1, w2):
    tok_per_w = T // _NW
    chunk = 32
    nchunks = tok_per_w // chunk
    hvecs = H // _L
    mesh = plsc.VectorSubcoreMesh(core_axis_name="c", subcore_axis_name="s")

    @functools.partial(
        pl.kernel, mesh=mesh,
        compiler_params=pltpu.CompilerParams(needs_layout_passes=False),
        out_type=jax.ShapeDtypeStruct((T, H), jnp.float32),
        scratch_types=[
            pltpu.VMEM((tok_per_w,), jnp.int32),
            pltpu.VMEM((tok_per_w,), jnp.int32),
            pltpu.VMEM((tok_per_w,), jnp.float32),
            pltpu.VMEM((tok_per_w,), jnp.float32),
            pltpu.VMEM((chunk, H), jnp.float32),
            pltpu.VMEM((chunk, H), jnp.float32),
            pltpu.SemaphoreType.DMA,
        ],
    )
    def k(os_hbm, p1_hbm, p2_hbm, w1_hbm, w2_hbm, out_hbm,
          i1_v, i2_v, w1_v, w2_v, b1_v, b2_v, sem):
        wid = lax.axis_index("s") * _NC + lax.axis_index("c")
        base = wid * tok_per_w
        pltpu.sync_copy(p1_hbm.at[pl.ds(base, tok_per_w)], i1_v)
        pltpu.sync_copy(p2_hbm.at[pl.ds(base, tok_per_w)], i2_v)
        pltpu.sync_copy(w1_hbm.at[pl.ds(base, tok_per_w)], w1_v)
        pltpu.sync_copy(w2_hbm.at[pl.ds(base, tok_per_w)], w2_v)

        def body(c, carry):
            pltpu.async_copy(
                os_hbm.at[i1_v.at[pl.ds(c * chunk, chunk)]], b1_v, sem
            ).wait()
            pltpu.async_copy(
                os_hbm.at[i2_v.at[pl.ds(c * chunk, chunk)]], b2_v, sem
            ).wait()

            def row(r, rc):
                ridx = jnp.full((_L,), c * chunk + r, jnp.int32)
                ws1 = plsc.load_gather(w1_v, [ridx])
                ws2 = plsc.load_gather(w2_v, [ridx])

                def add(j, jc):
                    slj = pl.ds(j * _L, _L)
                    b1_v[r, slj] = ws1 * b1_v[r, slj] + ws2 * b2_v[r, slj]
                    return jc

                lax.fori_loop(0, hvecs, add, 0, unroll=4)
                return rc

            lax.fori_loop(0, chunk, row, 0, unroll=False)
            pltpu.sync_copy(
                b1_v, out_hbm.at[pl.ds(base + c * chunk, chunk)]
            )
            return carry

        lax.fori_loop(0, nchunks, body, 0, unroll=False)

    return k(out_sorted, pos1, pos2, w1, w2)


def kernel(hidden_states, gate_weight, gate_up_weight, down_weight):
    x = hidden_states
    w1, w2, i1f, i2f, r1f, r2f, cnt8, meta = _router(x, gate_weight)

    xs, pos1, pos2, eid, used = _sc_dispatch(
        x, i1f.reshape(T), i2f.reshape(T), r1f.reshape(T), r2f.reshape(T), meta)
    out_sorted = _gmm(eid, used, xs, gate_up_weight, down_weight)
    return _sc_combine(out_sorted, pos1, pos2, w1.reshape(T), w2.reshape(T))
